# baseline (speedup 1.0000x reference)
import functools

import jax
import jax.numpy as jnp
from jax import lax
from jax.experimental import pallas as pl
from jax.experimental.pallas import tpu as pltpu
from jax.experimental.pallas import tpu_sc as plsc

SHIFT = -16
NEG_INF = float("-inf")
LANES = 16
NWIN = 7


def _window_bounds(leff):
    bounds = [(jnp.int32(0), leff)]
    for lvl in (1, 2):
        d = 2 ** lvl
        k = (leff + d - 1) // d
        s = leff // d
        for j in range(d):
            lo = jnp.int32(j) * s
            hi = jnp.minimum(lo + k, leff)
            bounds.append((lo, hi))
    return bounds


def _make_sc_kernel(B, C, L, interpret=False):
    ROWS = B * C
    NW = 32
    RPW = ROWS // NW
    NSPAN = L // 256
    NC = 2

    def body(x_hbm, lens_hbm, out_hbm, lens_v, buf0, buf1, sv_ref, r16_ref,
             out_buf, sem0, sem1):
        wid = lax.axis_index("s") * NC + lax.axis_index("c")
        row0 = wid * RPW
        sample = row0 // C

        pltpu.sync_copy(lens_hbm, lens_v)
        iota = lax.broadcasted_iota(jnp.int32, (LANES,), 0)
        lvec = plsc.load_gather(lens_v, [iota * 0 + sample])
        leff = jnp.minimum(lvec[0] - SHIFT, L)
        windows = _window_bounds(leff)

        penalties = []
        edge_bases = []
        for lo, hi in windows:
            penalties.append([
                jnp.where((s * 256 >= lo) & (s * 256 + 256 <= hi),
                          0.0, NEG_INF).astype(jnp.float32)
                for s in range(NSPAN)
            ])
            edge_bases.append(((lo // 256) * 256, ((hi - 1) // 256) * 256))

        def compute_row(buf, slot):
            for g in range(NSPAN):
                m = buf[pl.ds(g * 256, LANES)]
                for k in range(1, 16):
                    m = jnp.maximum(m, buf[pl.ds(g * 256 + k * LANES, LANES)])
                sv_ref[pl.ds(g * LANES, LANES)] = m

            res = jnp.full((LANES,), NEG_INF, jnp.float32)
            for w, (lo, hi) in enumerate(windows):
                acc = jnp.full((LANES,), NEG_INF, jnp.float32)
                for s in range(NSPAN):
                    acc = jnp.maximum(
                        acc, sv_ref[pl.ds(s * LANES, LANES)] + penalties[w][s])
                for base in edge_bases[w]:
                    for k in range(16):
                        v = buf[pl.ds(base + k * LANES, LANES)]
                        pos = iota + (base + k * LANES)
                        msk = (pos >= lo) & (pos < hi)
                        acc = jnp.maximum(acc, jnp.where(msk, v, NEG_INF))
                for stride in (8, 4, 2, 1):
                    r16_ref[...] = acc
                    acc = jnp.maximum(
                        acc, plsc.load_gather(r16_ref, [iota ^ stride]))
                res = jnp.where(iota == w, acc, res)
            out_buf[pl.ds(slot * LANES, LANES)] = res

        pltpu.async_copy(x_hbm.at[row0], buf0, sem0)
        pltpu.async_copy(x_hbm.at[row0 + 1], buf1, sem1)

        @pl.loop(0, RPW, step=2)
        def row_loop(g):
            for b, (buf, sem) in enumerate(((buf0, sem0), (buf1, sem1))):
                r = g + b
                pltpu.make_async_copy(x_hbm.at[0], buf, sem).wait()
                compute_row(buf, r)
                nxt = r + 2

                @pl.when(nxt < RPW)
                def _():
                    pltpu.async_copy(x_hbm.at[row0 + nxt], buf, sem)

        pltpu.sync_copy(out_buf, out_hbm.at[pl.ds(row0 * LANES, RPW * LANES)])

    return pl.kernel(
        body,
        out_type=jax.ShapeDtypeStruct((ROWS * LANES,), jnp.float32),
        mesh=plsc.VectorSubcoreMesh(core_axis_name="c", subcore_axis_name="s",
                                    num_cores=NC, num_subcores=16),
        compiler_params=pltpu.CompilerParams(needs_layout_passes=False),
        scratch_types=[
            pltpu.VMEM((LANES,), jnp.int32),
            pltpu.VMEM((L,), jnp.float32),
            pltpu.VMEM((L,), jnp.float32),
            pltpu.VMEM((NSPAN * LANES,), jnp.float32),
            pltpu.VMEM((LANES,), jnp.float32),
            pltpu.VMEM((RPW * LANES,), jnp.float32),
            pltpu.SemaphoreType.DMA,
            pltpu.SemaphoreType.DMA,
        ],
        interpret=interpret,
    )


def kernel(x, orig_len):
    B, C, L = x.shape
    lens = jnp.asarray(orig_len, jnp.int32)
    sc = _make_sc_kernel(B, C, L)
    out = sc(x.reshape(B * C, L), lens).reshape(B, C, LANES)
    return jnp.concatenate(
        [out[:, :, 0],
         out[:, :, 1:3].reshape(B, 2 * C),
         out[:, :, 3:7].reshape(B, 4 * C)], axis=1)


# speedup vs baseline: 1.2466x; 1.2466x over previous
import functools

import jax
import jax.numpy as jnp
from jax import lax
from jax.experimental import pallas as pl
from jax.experimental.pallas import tpu as pltpu
from jax.experimental.pallas import tpu_sc as plsc

SHIFT = -16
NEG_INF = float("-inf")
LANES = 16
NWIN = 7


def _window_bounds(leff):
    bounds = [(jnp.int32(0), leff)]
    for lvl in (1, 2):
        d = 2 ** lvl
        k = (leff + d - 1) // d
        s = leff // d
        for j in range(d):
            lo = jnp.int32(j) * s
            hi = jnp.minimum(lo + k, leff)
            bounds.append((lo, hi))
    return bounds


def _make_sc_kernel(B, C, L, interpret=False):
    ROWS = B * C
    NW = 32
    RPW = ROWS // NW
    NSPAN = L // 256
    NC = 2

    def body(x_hbm, lens_hbm, out_hbm, lens_v, buf0, buf1, sv_ref, sv64_ref,
             r16_ref, out_buf, sem0, sem1):
        wid = lax.axis_index("s") * NC + lax.axis_index("c")
        row0 = wid * RPW
        sample = row0 // C

        pltpu.sync_copy(lens_hbm, lens_v)
        iota = lax.broadcasted_iota(jnp.int32, (LANES,), 0)
        lvec = plsc.load_gather(lens_v, [iota * 0 + sample])
        leff = jnp.minimum(lvec[0] - SHIFT, L)
        windows = _window_bounds(leff)

        def pen(cond):
            return jnp.where(cond, 0.0, NEG_INF).astype(jnp.float32)

        def compute_row(buf, slot):
            for g in range(NSPAN):
                subs = []
                for q in range(4):
                    base = g * 256 + q * 64
                    m = buf[pl.ds(base, LANES)]
                    for k in range(1, 4):
                        m = jnp.maximum(m, buf[pl.ds(base + k * LANES, LANES)])
                    sv64_ref[pl.ds((g * 4 + q) * LANES, LANES)] = m
                    subs.append(m)
                sv_ref[pl.ds(g * LANES, LANES)] = jnp.maximum(
                    jnp.maximum(subs[0], subs[1]), jnp.maximum(subs[2], subs[3]))

            res = jnp.full((LANES,), NEG_INF, jnp.float32)
            for w, (lo, hi) in enumerate(windows):
                acc = jnp.full((LANES,), NEG_INF, jnp.float32)
                for s in range(NSPAN):
                    acc = jnp.maximum(
                        acc, sv_ref[pl.ds(s * LANES, LANES)]
                        + pen((s * 256 >= lo) & (s * 256 + 256 <= hi)))
                for ebase in ((lo // 256) * 256, ((hi - 1) // 256) * 256):
                    sub0 = (ebase // 64) * LANES
                    for q in range(4):
                        b = ebase + q * 64
                        acc = jnp.maximum(
                            acc, sv64_ref[pl.ds(sub0 + q * LANES, LANES)]
                            + pen((b >= lo) & (b + 64 <= hi)))
                for bbase in ((lo // 64) * 64, ((hi - 1) // 64) * 64):
                    pos0 = iota + bbase
                    for k in range(4):
                        v = buf[pl.ds(bbase + k * LANES, LANES)]
                        pos = pos0 + k * LANES
                        msk = (pos >= lo) & (pos < hi)
                        acc = jnp.maximum(acc, jnp.where(msk, v, NEG_INF))
                for stride in (8, 4, 2, 1):
                    r16_ref[...] = acc
                    acc = jnp.maximum(
                        acc, plsc.load_gather(r16_ref, [iota ^ stride]))
                res = jnp.where(iota == w, acc, res)
            out_buf[pl.ds(slot * LANES, LANES)] = res

        pltpu.async_copy(x_hbm.at[row0], buf0, sem0)
        pltpu.async_copy(x_hbm.at[row0 + 1], buf1, sem1)

        @pl.loop(0, RPW, step=2)
        def row_loop(g):
            for b, (buf, sem) in enumerate(((buf0, sem0), (buf1, sem1))):
                r = g + b
                pltpu.make_async_copy(x_hbm.at[0], buf, sem).wait()
                compute_row(buf, r)
                nxt = r + 2

                @pl.when(nxt < RPW)
                def _():
                    pltpu.async_copy(x_hbm.at[row0 + nxt], buf, sem)

        pltpu.sync_copy(out_buf, out_hbm.at[pl.ds(row0 * LANES, RPW * LANES)])

    return pl.kernel(
        body,
        out_type=jax.ShapeDtypeStruct((ROWS * LANES,), jnp.float32),
        mesh=plsc.VectorSubcoreMesh(core_axis_name="c", subcore_axis_name="s",
                                    num_cores=NC, num_subcores=16),
        compiler_params=pltpu.CompilerParams(needs_layout_passes=False),
        scratch_types=[
            pltpu.VMEM((LANES,), jnp.int32),
            pltpu.VMEM((L,), jnp.float32),
            pltpu.VMEM((L,), jnp.float32),
            pltpu.VMEM((NSPAN * LANES,), jnp.float32),
            pltpu.VMEM((NSPAN * 4 * LANES,), jnp.float32),
            pltpu.VMEM((LANES,), jnp.float32),
            pltpu.VMEM((RPW * LANES,), jnp.float32),
            pltpu.SemaphoreType.DMA,
            pltpu.SemaphoreType.DMA,
        ],
        interpret=interpret,
    )


def kernel(x, orig_len):
    B, C, L = x.shape
    lens = jnp.asarray(orig_len, jnp.int32)
    sc = _make_sc_kernel(B, C, L)
    out = sc(x.reshape(B * C, L), lens).reshape(B, C, LANES)
    return jnp.concatenate(
        [out[:, :, 0],
         out[:, :, 1:3].reshape(B, 2 * C),
         out[:, :, 3:7].reshape(B, 4 * C)], axis=1)


# speedup vs baseline: 1.3073x; 1.0487x over previous
import functools

import jax
import jax.numpy as jnp
from jax import lax
from jax.experimental import pallas as pl
from jax.experimental.pallas import tpu as pltpu
from jax.experimental.pallas import tpu_sc as plsc

SHIFT = -16
NEG_INF = float("-inf")
LANES = 16
NWIN = 7


def _window_bounds(leff):
    bounds = [(jnp.int32(0), leff)]
    for lvl in (1, 2):
        d = 2 ** lvl
        k = (leff + d - 1) // d
        s = leff // d
        for j in range(d):
            lo = jnp.int32(j) * s
            hi = jnp.minimum(lo + k, leff)
            bounds.append((lo, hi))
    return bounds


def _tree_max(terms):
    while len(terms) > 1:
        nxt = [jnp.maximum(a, b) for a, b in zip(terms[::2], terms[1::2])]
        if len(terms) % 2:
            nxt.append(terms[-1])
        terms = nxt
    return terms[0]


def _make_sc_kernel(B, C, L, interpret=False):
    ROWS = B * C
    NW = 32
    RPW = ROWS // NW
    NSPAN = L // 256
    NC = 2
    SB = 8
    NSB = RPW // SB

    def body(x_hbm, lens_hbm, out_hbm, lens_v, sbuf0, sbuf1, sv_ref, sv64_ref,
             scr_ref, out_buf, sem0, sem1):
        wid = lax.axis_index("s") * NC + lax.axis_index("c")
        row0 = wid * RPW
        sample = row0 // C

        pltpu.sync_copy(lens_hbm, lens_v)
        iota = lax.broadcasted_iota(jnp.int32, (LANES,), 0)
        lvec = plsc.load_gather(lens_v, [iota * 0 + sample])
        leff = jnp.minimum(lvec[0] - SHIFT, L)
        windows = _window_bounds(leff)
        k2 = (leff + 1) // 2
        iota16x = iota * LANES

        def pen(cond):
            return jnp.where(cond, 0.0, NEG_INF).astype(jnp.float32)

        def compute_row(buf, b0, slot):
            for g in range(NSPAN):
                subs = []
                for q in range(4):
                    base = b0 + g * 256 + q * 64
                    vs = [buf[pl.ds(base + k * LANES, LANES)]
                          for k in range(4)]
                    m = _tree_max(vs)
                    sv64_ref[pl.ds((g * 4 + q) * LANES, LANES)] = m
                    subs.append(m)
                sv_ref[pl.ds(g * LANES, LANES)] = _tree_max(subs)

            accs = [None] * NWIN
            for w in (2, 3, 4, 5, 6):
                lo, hi = windows[w]
                terms = []
                for s in range(NSPAN):
                    terms.append(
                        sv_ref[pl.ds(s * LANES, LANES)]
                        + pen((s * 256 >= lo) & (s * 256 + 256 <= hi)))
                for ebase in ((lo // 256) * 256, ((hi - 1) // 256) * 256):
                    sub0 = (ebase // 64) * LANES
                    for q in range(4):
                        b = ebase + q * 64
                        terms.append(
                            sv64_ref[pl.ds(sub0 + q * LANES, LANES)]
                            + pen((b >= lo) & (b + 64 <= hi)))
                for bbase in ((lo // 64) * 64, ((hi - 1) // 64) * 64):
                    pos0 = iota + bbase
                    for k in range(4):
                        v = buf[pl.ds(b0 + bbase + k * LANES, LANES)]
                        pos = pos0 + k * LANES
                        msk = (pos >= lo) & (pos < hi)
                        terms.append(jnp.where(msk, v, NEG_INF))
                acc = terms[0]
                for i in range(1, len(terms), 4):
                    acc = jnp.maximum(acc, _tree_max(terms[i:i + 4]))
                accs[w] = acc

            e1 = k2 - 1
            cbase = (e1 // LANES) * LANES
            v = buf[pl.ds(b0 + cbase, LANES)]
            corr = jnp.where(iota + cbase == e1, v, NEG_INF)
            accs[1] = jnp.maximum(jnp.maximum(accs[3], accs[4]), corr)
            accs[0] = jnp.maximum(accs[1], accs[2])

            for w in range(NWIN):
                scr_ref[pl.ds(w * LANES, LANES)] = accs[w]
            gathered = [plsc.load_gather(scr_ref, [iota16x + j])
                        for j in range(LANES)]
            out_buf[pl.ds(slot * LANES, LANES)] = _tree_max(gathered)

        def sb_src(sbi):
            return x_hbm.at[pl.ds((row0 + sbi * SB) * L, SB * L)]

        pltpu.async_copy(sb_src(0), sbuf0, sem0)
        pltpu.async_copy(sb_src(1), sbuf1, sem1)

        @pl.loop(0, NSB, step=2)
        def sb_loop(g):
            for b, (sbuf, sem) in enumerate(((sbuf0, sem0), (sbuf1, sem1))):
                sbi = g + b
                pltpu.make_async_copy(sb_src(0), sbuf, sem).wait()

                @pl.loop(0, SB)
                def row_loop(r):
                    compute_row(sbuf, r * L, sbi * SB + r)

                nxt = sbi + 2

                @pl.when(nxt < NSB)
                def _():
                    pltpu.async_copy(sb_src(nxt), sbuf, sem)

        pltpu.sync_copy(out_buf, out_hbm.at[pl.ds(row0 * LANES, RPW * LANES)])

    return pl.kernel(
        body,
        out_type=jax.ShapeDtypeStruct((ROWS * LANES,), jnp.float32),
        mesh=plsc.VectorSubcoreMesh(core_axis_name="c", subcore_axis_name="s",
                                    num_cores=NC, num_subcores=16),
        compiler_params=pltpu.CompilerParams(needs_layout_passes=False),
        scratch_types=[
            pltpu.VMEM((LANES,), jnp.int32),
            pltpu.VMEM((SB * L,), jnp.float32),
            pltpu.VMEM((SB * L,), jnp.float32),
            pltpu.VMEM((NSPAN * LANES,), jnp.float32),
            pltpu.VMEM((NSPAN * 4 * LANES,), jnp.float32),
            pltpu.VMEM((LANES * LANES,), jnp.float32),
            pltpu.VMEM((RPW * LANES,), jnp.float32),
            pltpu.SemaphoreType.DMA,
            pltpu.SemaphoreType.DMA,
        ],
        interpret=interpret,
    )


def kernel(x, orig_len):
    B, C, L = x.shape
    lens = jnp.asarray(orig_len, jnp.int32)
    sc = _make_sc_kernel(B, C, L)
    out = sc(x.reshape(B * C * L), lens).reshape(B, C, LANES)
    return jnp.concatenate(
        [out[:, :, 0],
         out[:, :, 1:3].reshape(B, 2 * C),
         out[:, :, 3:7].reshape(B, 4 * C)], axis=1)
